# Initial kernel scaffold; baseline (speedup 1.0000x reference)
import jax
import jax.numpy as jnp
from jax.experimental import pallas as pl


def kernel(x, latency_to_accumulate, AP_path_alpha, gammas, betas):
    raise NotImplementedError("write your pallas kernel here")


# speedup vs baseline: 1.0095x; 1.0095x over previous
import functools

import jax
import jax.numpy as jnp
from jax import lax
from jax.experimental import pallas as pl

E = 8
D = 2048
N = 32768
BLOCK_N = 512


def _affine_body(x_ref, lat0_ref, alpha_ref, g_ref, b_ref, out_ref, lat_ref):
    out_ref[...] = x_ref[...] * g_ref[...] + b_ref[...]

    @pl.when(pl.program_id(0) == 0)
    def _():
        a = alpha_ref[...]
        m = jnp.max(a)
        e = jnp.exp(a - m)
        i = lax.broadcasted_iota(jnp.int32, (1, E), 1).astype(jnp.float32)
        latency = 0.5 + i * (1.5 / (E - 1))
        lat_ref[...] = lat0_ref[...] + jnp.sum(latency * e) / jnp.sum(e)


@jax.jit
def kernel(x, latency_to_accumulate, AP_path_alpha, gammas, betas):
    grid = (N // BLOCK_N,)
    out, lat = pl.pallas_call(
        _affine_body,
        grid=grid,
        in_specs=[
            pl.BlockSpec((BLOCK_N, D), lambda i: (i, 0)),
            pl.BlockSpec((1, 1), lambda i: (0, 0)),
            pl.BlockSpec((1, E), lambda i: (0, 0)),
            pl.BlockSpec((1, D), lambda i: (0, 0)),
            pl.BlockSpec((1, D), lambda i: (0, 0)),
        ],
        out_specs=[
            pl.BlockSpec((BLOCK_N, D), lambda i: (i, 0)),
            pl.BlockSpec((1, 1), lambda i: (0, 0)),
        ],
        out_shape=[
            jax.ShapeDtypeStruct((N, D), jnp.float32),
            jax.ShapeDtypeStruct((1, 1), jnp.float32),
        ],
    )(
        x,
        latency_to_accumulate.reshape(1, 1),
        AP_path_alpha.reshape(1, E),
        gammas[0:1],
        betas[0:1],
    )
    return out, lat.reshape(())


# speedup vs baseline: 1.0227x; 1.0131x over previous
import functools

import jax
import jax.numpy as jnp
from jax import lax
from jax.experimental import pallas as pl

E = 8
D = 2048
N = 32768
BLOCK_N = 1024


def _affine_body(x_ref, lat0_ref, alpha_ref, g_ref, b_ref, out_ref, lat_ref):
    out_ref[...] = x_ref[...] * g_ref[...] + b_ref[...]

    @pl.when(pl.program_id(0) == 0)
    def _():
        a = alpha_ref[...]
        m = jnp.max(a)
        e = jnp.exp(a - m)
        i = lax.broadcasted_iota(jnp.int32, (1, E), 1).astype(jnp.float32)
        latency = 0.5 + i * (1.5 / (E - 1))
        lat_ref[...] = lat0_ref[...] + jnp.sum(latency * e) / jnp.sum(e)


@jax.jit
def kernel(x, latency_to_accumulate, AP_path_alpha, gammas, betas):
    grid = (N // BLOCK_N,)
    out, lat = pl.pallas_call(
        _affine_body,
        grid=grid,
        in_specs=[
            pl.BlockSpec((BLOCK_N, D), lambda i: (i, 0)),
            pl.BlockSpec((1, 1), lambda i: (0, 0)),
            pl.BlockSpec((1, E), lambda i: (0, 0)),
            pl.BlockSpec((1, D), lambda i: (0, 0)),
            pl.BlockSpec((1, D), lambda i: (0, 0)),
        ],
        out_specs=[
            pl.BlockSpec((BLOCK_N, D), lambda i: (i, 0)),
            pl.BlockSpec((1, 1), lambda i: (0, 0)),
        ],
        out_shape=[
            jax.ShapeDtypeStruct((N, D), jnp.float32),
            jax.ShapeDtypeStruct((1, 1), jnp.float32),
        ],
    )(
        x,
        latency_to_accumulate.reshape(1, 1),
        AP_path_alpha.reshape(1, E),
        gammas[0:1],
        betas[0:1],
    )
    return out, lat.reshape(())
